# baseline (speedup 1.0000x reference)
import functools

import jax
import jax.numpy as jnp
from jax import lax
from jax.experimental import pallas as pl
from jax.experimental.pallas import tpu as pltpu
from jax.experimental.pallas import tpu_sc as plsc

_NC = 2
_NS = 16


def _make_sc_copy(n):
    nw = _NC * _NS
    per_w = n // nw
    mesh = plsc.VectorSubcoreMesh(core_axis_name="c", subcore_axis_name="s")

    @functools.partial(
        pl.kernel,
        mesh=mesh,
        out_type=jax.ShapeDtypeStruct((n,), jnp.float32),
        scratch_types=[
            pltpu.VMEM((per_w,), jnp.float32),
        ],
    )
    def copy_k(q_hbm, out_hbm, buf):
        wid = lax.axis_index("s") * _NC + lax.axis_index("c")
        base = wid * per_w
        pltpu.sync_copy(q_hbm.at[pl.ds(base, per_w)], buf)
        pltpu.sync_copy(buf, out_hbm.at[pl.ds(base, per_w)])

    return copy_k


def kernel(c, q, emb_table):
    del c, emb_table
    rows, cols = q.shape
    n = rows * cols
    qf = q.T.reshape(n)
    out = _make_sc_copy(n)(qf)
    return out.reshape(cols, rows).T
